# Initial kernel scaffold; baseline (speedup 1.0000x reference)
import jax
import jax.numpy as jnp
from jax.experimental import pallas as pl


def kernel(x, edge_index_temp, edge_index_intersects, params):
    raise NotImplementedError("write your pallas kernel here")


# speedup vs baseline: 5.7809x; 5.7809x over previous
import functools

import jax
import jax.numpy as jnp
from jax import lax
from jax.experimental import pallas as pl
from jax.experimental.pallas import tpu as pltpu
from jax.experimental.pallas import tpu_sc as plsc

NC = 2
NS = 16
WQ = 8
CHUNK = 128
U = 8
ZR = 1280
DUMP = 2400

_mesh = plsc.VectorSubcoreMesh(core_axis_name="c", subcore_axis_name="s",
                               num_cores=NC, num_subcores=NS)


def _fill_rows(ref, nrows, value):
  def body(k, _):
    ref[k] = jnp.full((WQ,), value, jnp.float32)
    return 0
  lax.fori_loop(0, nrows, body, 0, unroll=8)


def _sc_layer_fn(n_nodes, table_rows, nblk_t, nblk_i):
  acc_rows = n_nodes + DUMP
  assert acc_rows % (NS * ZR) == 0
  zpt = acc_rows // (NS * ZR)
  assert n_nodes % NS == 0
  fpt = n_nodes // NS

  @functools.partial(
      pl.kernel,
      out_type=[jax.ShapeDtypeStruct((n_nodes, 4, WQ), jnp.float32)] * 2,
      mesh=_mesh,
      compiler_params=pltpu.CompilerParams(use_tc_tiling_on_sc=False),
      scratch_types=[
          pltpu.VMEM((U, CHUNK), jnp.int32),
          pltpu.VMEM((U, CHUNK), jnp.int32),
          pltpu.VMEM((U, CHUNK, WQ), jnp.float32),
          pltpu.VMEM((ZR, WQ), jnp.float32),
          pltpu.VMEM_SHARED((acc_rows, WQ), jnp.float32),
          pltpu.SemaphoreType.DMA,
      ],
  )
  def sc_layer(table, gidx_t, dst_t, gidx_i, dst_i, out_t, out_i,
               srcv, dstv, rows, zbuf, acc, sem):
    c = lax.axis_index("c")
    s = lax.axis_index("s")

    def zero_acc():
      for k in range(zpt):
        pltpu.sync_copy(zbuf, acc.at[pl.ds((s * zpt + k) * ZR, ZR)])

    def sweep(gidx, dst, nblk, p):
      cpt = nblk * U

      def body(b, _):
        row0 = s * cpt + b * U
        pltpu.sync_copy(gidx.at[c, p, pl.ds(row0, U)], srcv)
        pltpu.sync_copy(dst.at[pl.ds(row0, U)], dstv)
        cps = [pltpu.async_copy(table.at[srcv.at[j]], rows.at[j], sem)
               for j in range(U)]
        for cp in cps:
          cp.wait()
        for j in range(U):
          pltpu.sync_copy(rows.at[j], acc.at[dstv.at[j]], add=True)
        return 0

      lax.fori_loop(0, nblk, body, 0)

    def quarter_pass(gidx, dst, nblk, p, out):
      zero_acc()
      plsc.subcore_barrier()
      sweep(gidx, dst, nblk, p)
      plsc.subcore_barrier()
      pltpu.sync_copy(acc.at[pl.ds(s * fpt, fpt)],
                      out.at[pl.ds(s * fpt, fpt), 2 * c + p])
      plsc.subcore_barrier()

    _fill_rows(zbuf, ZR, 0.0)
    for p in range(2):
      quarter_pass(gidx_t, dst_t, nblk_t, p, out_t)
    for p in range(2):
      quarter_pass(gidx_i, dst_i, nblk_i, p, out_i)

  return sc_layer


def _sc_count_fn(n_nodes, nblk):
  acc_rows = n_nodes + DUMP
  zpt = acc_rows // (NS * ZR)
  fpt = n_nodes // NS

  @functools.partial(
      pl.kernel,
      out_type=jax.ShapeDtypeStruct((n_nodes, NC, WQ), jnp.float32),
      mesh=_mesh,
      compiler_params=pltpu.CompilerParams(use_tc_tiling_on_sc=False),
      scratch_types=[
          pltpu.VMEM((U, CHUNK), jnp.int32),
          pltpu.VMEM((CHUNK, WQ), jnp.float32),
          pltpu.VMEM((ZR, WQ), jnp.float32),
          pltpu.VMEM_SHARED((acc_rows, WQ), jnp.float32),
      ],
  )
  def sc_count(dst_i, out, dstv, ones, zbuf, acc):
    c = lax.axis_index("c")
    s = lax.axis_index("s")

    _fill_rows(zbuf, ZR, 0.0)
    _fill_rows(ones, CHUNK, 1.0)
    for k in range(zpt):
      pltpu.sync_copy(zbuf, acc.at[pl.ds((s * zpt + k) * ZR, ZR)])
    plsc.subcore_barrier()

    cpt = nblk * U

    def body(b, _):
      row0 = s * cpt + b * U
      pltpu.sync_copy(dst_i.at[pl.ds(row0, U)], dstv)
      for j in range(U):
        pltpu.sync_copy(ones, acc.at[dstv.at[j]], add=True)
      return 0

    lax.fori_loop(0, nblk, body, 0)
    plsc.subcore_barrier()
    pltpu.sync_copy(acc.at[pl.ds(s * fpt, fpt)],
                    out.at[pl.ds(s * fpt, fpt), c])

  return sc_count


_TCB = 2000


def _tc_proj0(x, wt, wi):
  n = x.shape[0]

  def body(x_ref, wt_ref, wi_ref, o_ref):
    xb = x_ref[...]
    o_ref[0] = jnp.dot(xb, wt_ref[...], preferred_element_type=jnp.float32)
    o_ref[1] = jnp.dot(xb, wi_ref[...], preferred_element_type=jnp.float32)

  return pl.pallas_call(
      body,
      grid=(n // _TCB,),
      in_specs=[
          pl.BlockSpec((_TCB, 128), lambda i: (i, 0)),
          pl.BlockSpec((128, 32), lambda i: (0, 0)),
          pl.BlockSpec((128, 32), lambda i: (0, 0)),
      ],
      out_specs=pl.BlockSpec((2, _TCB, 32), lambda i: (0, i, 0)),
      out_shape=jax.ShapeDtypeStruct((2, n, 32), jnp.float32),
  )(x, wt, wi)


def _tc_combine0(st, si, cnt, bt, bi):
  n = st.shape[0]

  def body(st_ref, si_ref, cnt_ref, bt_ref, bi_ref, o_ref):
    scale = 1.0 / jnp.maximum(cnt_ref[...], 1.0)
    o = st_ref[...] + si_ref[...] * scale + bt_ref[...] + bi_ref[...]
    o_ref[...] = jnp.maximum(o, 0.0)

  return pl.pallas_call(
      body,
      grid=(n // _TCB,),
      in_specs=[
          pl.BlockSpec((_TCB, 32), lambda i: (i, 0)),
          pl.BlockSpec((_TCB, 32), lambda i: (i, 0)),
          pl.BlockSpec((_TCB, 1), lambda i: (i, 0)),
          pl.BlockSpec((1, 32), lambda i: (0, 0)),
          pl.BlockSpec((1, 32), lambda i: (0, 0)),
      ],
      out_specs=pl.BlockSpec((_TCB, 32), lambda i: (i, 0)),
      out_shape=jax.ShapeDtypeStruct((n, 32), jnp.float32),
  )(st, si, cnt, bt.reshape(1, 32), bi.reshape(1, 32))


def _tc_combine(st, si, cnt, wt, wi, bt, bi, hprev):
  n = st.shape[0]
  dout = wt.shape[1]
  residual = hprev is not None

  def body(st_ref, si_ref, cnt_ref, wt_ref, wi_ref, bt_ref, bi_ref,
           *rest):
    if residual:
      hp_ref, o_ref = rest
    else:
      (o_ref,) = rest
    scale = 1.0 / jnp.maximum(cnt_ref[...], 1.0)
    mi = si_ref[...] * scale
    o = (jnp.dot(st_ref[...], wt_ref[...], preferred_element_type=jnp.float32)
         + jnp.dot(mi, wi_ref[...], preferred_element_type=jnp.float32)
         + bt_ref[...] + bi_ref[...])
    if residual:
      o = o + hp_ref[...]
    o_ref[...] = jnp.maximum(o, 0.0)

  in_specs = [
      pl.BlockSpec((_TCB, 32), lambda i: (i, 0)),
      pl.BlockSpec((_TCB, 32), lambda i: (i, 0)),
      pl.BlockSpec((_TCB, 1), lambda i: (i, 0)),
      pl.BlockSpec((32, dout), lambda i: (0, 0)),
      pl.BlockSpec((32, dout), lambda i: (0, 0)),
      pl.BlockSpec((1, dout), lambda i: (0, 0)),
      pl.BlockSpec((1, dout), lambda i: (0, 0)),
  ]
  args = [st, si, cnt, wt, wi, bt.reshape(1, dout), bi.reshape(1, dout)]
  if residual:
    in_specs.append(pl.BlockSpec((_TCB, 32), lambda i: (i, 0)))
    args.append(hprev)
  return pl.pallas_call(
      body,
      grid=(n // _TCB,),
      in_specs=in_specs,
      out_specs=pl.BlockSpec((_TCB, dout), lambda i: (i, 0)),
      out_shape=jax.ShapeDtypeStruct((n, dout), jnp.float32),
  )(*args)


def _prep_edges(src, dst, n_nodes, base):
  e = src.shape[0]
  blk = NS * U * CHUNK
  p = ((e + blk - 1) // blk) * blk
  pad = p - e
  fill = jnp.arange(pad, dtype=jnp.int32)
  src_p = jnp.concatenate([src, fill % n_nodes])
  dst_p = jnp.concatenate([dst, n_nodes + (fill % DUMP)])
  g = base + 4 * src_p
  gidx = jnp.stack([g, g + 1, g + 2, g + 3]).reshape(2, 2, p // CHUNK, CHUNK)
  return gidx, dst_p.reshape(p // CHUNK, CHUNK), p // blk


def kernel(x, edge_index_temp, edge_index_intersects, params):
  n = x.shape[0]
  src_t, dst_t = edge_index_temp[0], edge_index_temp[1]
  src_i, dst_i = edge_index_intersects[0], edge_index_intersects[1]

  gidx_t, dstp_t, nblk_t = _prep_edges(src_t, dst_t, n, 0)
  gidx_i, dstp_i, nblk_i = _prep_edges(src_i, dst_i, n, 0)
  gidx_i0 = gidx_i + 4 * n

  sc_count = _sc_count_fn(n, nblk_i)
  sc_layer0 = _sc_layer_fn(n, 8 * n, nblk_t, nblk_i)
  sc_layer = _sc_layer_fn(n, 4 * n, nblk_t, nblk_i)

  cnt = sc_count(dstp_i)[:, 0, :1]

  p0 = params[0]
  table0 = _tc_proj0(x, p0["Wt"], p0["Wi"]).reshape(8 * n, WQ)
  st, si = sc_layer0(table0, gidx_t, dstp_t, gidx_i0, dstp_i)
  h = _tc_combine0(st.reshape(n, 32), si.reshape(n, 32), cnt,
                   p0["bt"], p0["bi"])

  for p in params[1:]:
    st, si = sc_layer(h.reshape(4 * n, WQ), gidx_t, dstp_t, gidx_i, dstp_i)
    hprev = h if p["Wt"].shape[1] == h.shape[1] else None
    h = _tc_combine(st.reshape(n, 32), si.reshape(n, 32), cnt,
                    p["Wt"], p["Wi"], p["bt"], p["bi"], hprev)
  return h


# speedup vs baseline: 9.4736x; 1.6388x over previous
import functools

import jax
import jax.numpy as jnp
from jax import lax
from jax.experimental import pallas as pl
from jax.experimental.pallas import tpu as pltpu
from jax.experimental.pallas import tpu_sc as plsc

NC = 2
NS = 16
WQ = 8
CHUNK = 128
U = 8
ZR = 1280
DUMP = 2400

_mesh = plsc.VectorSubcoreMesh(core_axis_name="c", subcore_axis_name="s",
                               num_cores=NC, num_subcores=NS)


def _fill_rows(ref, nrows, value):
  def body(k, _):
    ref[k] = jnp.full((WQ,), value, jnp.float32)
    return 0
  lax.fori_loop(0, nrows, body, 0, unroll=8)


def _sc_layer_fn(n_nodes, table_rows, nblk_t, nblk_i):
  acc_rows = n_nodes + DUMP
  assert acc_rows % (NS * ZR) == 0
  zpt = acc_rows // (NS * ZR)
  assert n_nodes % NS == 0
  fpt = n_nodes // NS

  @functools.partial(
      pl.kernel,
      out_type=[jax.ShapeDtypeStruct((n_nodes, 4, WQ), jnp.float32)] * 2,
      mesh=_mesh,
      compiler_params=pltpu.CompilerParams(use_tc_tiling_on_sc=False),
      scratch_types=[
          pltpu.VMEM((U, CHUNK), jnp.int32),
          pltpu.VMEM((U, CHUNK), jnp.int32),
          pltpu.VMEM((U, CHUNK, WQ), jnp.float32),
          pltpu.VMEM((ZR, WQ), jnp.float32),
          pltpu.VMEM_SHARED((acc_rows, WQ), jnp.float32),
          pltpu.SemaphoreType.DMA,
      ],
  )
  def sc_layer(table, gidx_t, dst_t, gidx_i, dst_i, out_t, out_i,
               srcv, dstv, rows, zbuf, acc, sem):
    c = lax.axis_index("c")
    s = lax.axis_index("s")

    def zero_acc():
      for k in range(zpt):
        pltpu.sync_copy(zbuf, acc.at[pl.ds((s * zpt + k) * ZR, ZR)])

    def sweep(gidx, dst, nblk, p):
      cpt = nblk * U

      def body(b, _):
        row0 = s * cpt + b * U
        pltpu.sync_copy(gidx.at[c, p, pl.ds(row0, U)], srcv)
        pltpu.sync_copy(dst.at[pl.ds(row0, U)], dstv)
        cps = [pltpu.async_copy(table.at[srcv.at[j]], rows.at[j], sem)
               for j in range(U)]
        for cp in cps:
          cp.wait()
        for j in range(U):
          pltpu.sync_copy(rows.at[j], acc.at[dstv.at[j]], add=True)
        return 0

      lax.fori_loop(0, nblk, body, 0)

    def quarter_pass(gidx, dst, nblk, p, out):
      zero_acc()
      plsc.subcore_barrier()
      sweep(gidx, dst, nblk, p)
      plsc.subcore_barrier()
      pltpu.sync_copy(acc.at[pl.ds(s * fpt, fpt)],
                      out.at[pl.ds(s * fpt, fpt), 2 * c + p])
      plsc.subcore_barrier()

    _fill_rows(zbuf, ZR, 0.0)
    for p in range(2):
      quarter_pass(gidx_t, dst_t, nblk_t, p, out_t)
    for p in range(2):
      quarter_pass(gidx_i, dst_i, nblk_i, p, out_i)

  return sc_layer


def _sc_count_fn(n_nodes, nblk):
  acc_rows = n_nodes + DUMP
  zpt = acc_rows // (NS * ZR)
  fpt = n_nodes // NS

  @functools.partial(
      pl.kernel,
      out_type=jax.ShapeDtypeStruct((n_nodes, 4, WQ), jnp.float32),
      mesh=_mesh,
      compiler_params=pltpu.CompilerParams(use_tc_tiling_on_sc=False),
      scratch_types=[
          pltpu.VMEM((U, CHUNK), jnp.int32),
          pltpu.VMEM((CHUNK, WQ), jnp.float32),
          pltpu.VMEM((ZR, WQ), jnp.float32),
          pltpu.VMEM_SHARED((acc_rows, WQ), jnp.float32),
      ],
  )
  def sc_count(dst_i, out, dstv, ones, zbuf, acc):
    c = lax.axis_index("c")
    s = lax.axis_index("s")

    _fill_rows(zbuf, ZR, 0.0)
    _fill_rows(ones, CHUNK, 1.0)
    for k in range(zpt):
      pltpu.sync_copy(zbuf, acc.at[pl.ds((s * zpt + k) * ZR, ZR)])
    plsc.subcore_barrier()

    cpt = nblk * U

    def body(b, _):
      row0 = s * cpt + b * U
      pltpu.sync_copy(dst_i.at[pl.ds(row0, U)], dstv)
      for j in range(U):
        pltpu.sync_copy(ones, acc.at[dstv.at[j]], add=True)
      return 0

    lax.fori_loop(0, nblk, body, 0)
    plsc.subcore_barrier()
    for p in range(2):
      pltpu.sync_copy(acc.at[pl.ds(s * fpt, fpt)],
                      out.at[pl.ds(s * fpt, fpt), 2 * c + p])

  return sc_count


_TCB = 4000
_PB = _TCB // 4


def _tc_proj0(xp, bdt, bdi):
  m = xp.shape[0]

  def body(x_ref, wt_ref, wi_ref, o_ref):
    xb = x_ref[...]
    o_ref[0] = jnp.dot(xb, wt_ref[...], preferred_element_type=jnp.float32)
    o_ref[1] = jnp.dot(xb, wi_ref[...], preferred_element_type=jnp.float32)

  return pl.pallas_call(
      body,
      grid=(m // _PB,),
      in_specs=[
          pl.BlockSpec((_PB, 512), lambda i: (i, 0)),
          pl.BlockSpec((512, 128), lambda i: (0, 0)),
          pl.BlockSpec((512, 128), lambda i: (0, 0)),
      ],
      out_specs=pl.BlockSpec((2, _PB, 128), lambda i: (0, i, 0)),
      out_shape=jax.ShapeDtypeStruct((2, m, 128), jnp.float32),
  )(xp, bdt, bdi)


def _tc_combine0(st, si, cntp, btp, bip):
  m = st.shape[0]

  def body(st_ref, si_ref, cnt_ref, bt_ref, bi_ref, o_ref):
    scale = 1.0 / jnp.maximum(cnt_ref[...], 1.0)
    o = st_ref[...] + si_ref[...] * scale + bt_ref[...] + bi_ref[...]
    o_ref[...] = jnp.maximum(o, 0.0)

  return pl.pallas_call(
      body,
      grid=(m // _PB,),
      in_specs=[
          pl.BlockSpec((_PB, 128), lambda i: (i, 0)),
          pl.BlockSpec((_PB, 128), lambda i: (i, 0)),
          pl.BlockSpec((_PB, 128), lambda i: (i, 0)),
          pl.BlockSpec((1, 128), lambda i: (0, 0)),
          pl.BlockSpec((1, 128), lambda i: (0, 0)),
      ],
      out_specs=pl.BlockSpec((_PB, 128), lambda i: (i, 0)),
      out_shape=jax.ShapeDtypeStruct((m, 128), jnp.float32),
  )(st, si, cntp, btp, bip)


def _tc_combine(st, si, cntp, bdt, bdi, btp, bip, hprev):
  m = st.shape[0]
  dcols = bdt.shape[1]
  residual = hprev is not None

  def body(st_ref, si_ref, cnt_ref, wt_ref, wi_ref, bt_ref, bi_ref, *rest):
    if residual:
      hp_ref, o_ref = rest
    else:
      (o_ref,) = rest
    scale = 1.0 / jnp.maximum(cnt_ref[...], 1.0)
    mi = si_ref[...] * scale
    o = (jnp.dot(st_ref[...], wt_ref[...], preferred_element_type=jnp.float32)
         + jnp.dot(mi, wi_ref[...], preferred_element_type=jnp.float32)
         + bt_ref[...] + bi_ref[...])
    if residual:
      o = o + hp_ref[...]
    o_ref[...] = jnp.maximum(o, 0.0)

  in_specs = [
      pl.BlockSpec((_PB, 128), lambda i: (i, 0)),
      pl.BlockSpec((_PB, 128), lambda i: (i, 0)),
      pl.BlockSpec((_PB, 128), lambda i: (i, 0)),
      pl.BlockSpec((128, dcols), lambda i: (0, 0)),
      pl.BlockSpec((128, dcols), lambda i: (0, 0)),
      pl.BlockSpec((1, dcols), lambda i: (0, 0)),
      pl.BlockSpec((1, dcols), lambda i: (0, 0)),
  ]
  args = [st, si, cntp, bdt, bdi, btp, bip]
  if residual:
    in_specs.append(pl.BlockSpec((_PB, 128), lambda i: (i, 0)))
    args.append(hprev)
  return pl.pallas_call(
      body,
      grid=(m // _PB,),
      in_specs=in_specs,
      out_specs=pl.BlockSpec((_PB, dcols), lambda i: (i, 0)),
      out_shape=jax.ShapeDtypeStruct((m, dcols), jnp.float32),
  )(*args)


def _prep_edges(src, dst, n_nodes, base):
  e = src.shape[0]
  blk = NS * U * CHUNK
  p = ((e + blk - 1) // blk) * blk
  pad = p - e
  fill = jnp.arange(pad, dtype=jnp.int32)
  src_p = jnp.concatenate([src, fill % n_nodes])
  dst_p = jnp.concatenate([dst, n_nodes + (fill % DUMP)])
  g = base + 4 * src_p
  gidx = jnp.stack([g, g + 1, g + 2, g + 3]).reshape(2, 2, p // CHUNK, CHUNK)
  return gidx, dst_p.reshape(p // CHUNK, CHUNK), p // blk


def _blockdiag4(w):
  din, dout = w.shape
  bd = jnp.zeros((4, din, 4, dout), jnp.float32)
  for k in range(4):
    bd = bd.at[k, :, k, :].set(w)
  return bd.reshape(4 * din, 4 * dout)


def kernel(x, edge_index_temp, edge_index_intersects, params):
  n = x.shape[0]
  src_t, dst_t = edge_index_temp[0], edge_index_temp[1]
  src_i, dst_i = edge_index_intersects[0], edge_index_intersects[1]

  gidx_t, dstp_t, nblk_t = _prep_edges(src_t, dst_t, n, 0)
  gidx_i, dstp_i, nblk_i = _prep_edges(src_i, dst_i, n, 0)
  gidx_i0 = gidx_i + 4 * n

  sc_count = _sc_count_fn(n, nblk_i)
  sc_layer0 = _sc_layer_fn(n, 8 * n, nblk_t, nblk_i)
  sc_layer = _sc_layer_fn(n, 4 * n, nblk_t, nblk_i)

  cntp = sc_count(dstp_i).reshape(n // 4, 128)

  p0 = params[0]
  xp = x.reshape(n // 4, 512)
  table0 = _tc_proj0(xp, _blockdiag4(p0["Wt"]),
                     _blockdiag4(p0["Wi"])).reshape(8 * n, WQ)
  st, si = sc_layer0(table0, gidx_t, dstp_t, gidx_i0, dstp_i)
  h = _tc_combine0(st.reshape(n // 4, 128), si.reshape(n // 4, 128), cntp,
                   jnp.tile(p0["bt"], 4).reshape(1, 128),
                   jnp.tile(p0["bi"], 4).reshape(1, 128))

  for p in params[1:]:
    st, si = sc_layer(h.reshape(4 * n, WQ), gidx_t, dstp_t, gidx_i, dstp_i)
    dout = p["Wt"].shape[1]
    hprev = h if dout == 32 else None
    h = _tc_combine(st.reshape(n // 4, 128), si.reshape(n // 4, 128), cntp,
                    _blockdiag4(p["Wt"]), _blockdiag4(p["Wi"]),
                    jnp.tile(p["bt"], 4).reshape(1, 4 * dout),
                    jnp.tile(p["bi"], 4).reshape(1, 4 * dout), hprev)
  return h.reshape(n, 64)


# speedup vs baseline: 10.3018x; 1.0874x over previous
import functools

import jax
import jax.numpy as jnp
from jax import lax
from jax.experimental import pallas as pl
from jax.experimental.pallas import tpu as pltpu
from jax.experimental.pallas import tpu_sc as plsc

NC = 2
NS = 16
WQ = 8
CHUNK = 128
U = 8
ZR = 1280
DUMP = 2400

_mesh = plsc.VectorSubcoreMesh(core_axis_name="c", subcore_axis_name="s",
                               num_cores=NC, num_subcores=NS)


def _fill_rows(ref, nrows, value):
  def body(k, _):
    ref[k] = jnp.full((WQ,), value, jnp.float32)
    return 0
  lax.fori_loop(0, nrows, body, 0, unroll=8)


def _sc_layer_fn(n_nodes, table_rows, nblk_t, nblk_i):
  acc_rows = n_nodes + DUMP
  assert acc_rows % (NS * ZR) == 0
  zpt = acc_rows // (NS * ZR)
  assert n_nodes % NS == 0
  fpt = n_nodes // NS

  @functools.partial(
      pl.kernel,
      out_type=[jax.ShapeDtypeStruct((n_nodes, 4, WQ), jnp.float32)] * 2,
      mesh=_mesh,
      compiler_params=pltpu.CompilerParams(use_tc_tiling_on_sc=False),
      scratch_types=[
          pltpu.VMEM((U, CHUNK), jnp.int32),
          pltpu.VMEM((U, CHUNK), jnp.int32),
          pltpu.VMEM((U, CHUNK, WQ), jnp.float32),
          pltpu.VMEM((ZR, WQ), jnp.float32),
          pltpu.VMEM_SHARED((acc_rows, WQ), jnp.float32),
          pltpu.SemaphoreType.DMA,
          pltpu.SemaphoreType.DMA,
      ],
  )
  def sc_layer(table, gidx_t, dst_t, gidx_i, dst_i, out_t, out_i,
               srcv, dstv, rows, zbuf, acc, sem, sem_sc):
    c = lax.axis_index("c")
    s = lax.axis_index("s")

    def zero_acc():
      for k in range(zpt):
        pltpu.sync_copy(zbuf, acc.at[pl.ds((s * zpt + k) * ZR, ZR)])

    def sweep(gidx, dst, nblk, p):
      cpt = nblk * U

      def drain_scatters():
        for j in range(U):
          pltpu.make_async_copy(rows.at[j], acc.at[dstv.at[j]],
                                sem_sc).wait()

      def body(b, _):
        row0 = s * cpt + b * U

        @pl.when(b > 0)
        def _():
          drain_scatters()

        pltpu.sync_copy(gidx.at[c, p, pl.ds(row0, U)], srcv)
        pltpu.sync_copy(dst.at[pl.ds(row0, U)], dstv)
        cps = [pltpu.async_copy(table.at[srcv.at[j]], rows.at[j], sem)
               for j in range(U)]
        for cp in cps:
          cp.wait()
        for j in range(U):
          pltpu.async_copy(rows.at[j], acc.at[dstv.at[j]], sem_sc, add=True)
        return 0

      lax.fori_loop(0, nblk, body, 0)
      drain_scatters()

    def quarter_pass(gidx, dst, nblk, p, out):
      zero_acc()
      plsc.subcore_barrier()
      sweep(gidx, dst, nblk, p)
      plsc.subcore_barrier()
      pltpu.sync_copy(acc.at[pl.ds(s * fpt, fpt)],
                      out.at[pl.ds(s * fpt, fpt), 2 * c + p])
      plsc.subcore_barrier()

    _fill_rows(zbuf, ZR, 0.0)
    for p in range(2):
      quarter_pass(gidx_t, dst_t, nblk_t, p, out_t)
    for p in range(2):
      quarter_pass(gidx_i, dst_i, nblk_i, p, out_i)

  return sc_layer


def _sc_count_fn(n_nodes, nblk):
  acc_rows = n_nodes + DUMP
  zpt = acc_rows // (NS * ZR)
  fpt = n_nodes // NS

  @functools.partial(
      pl.kernel,
      out_type=jax.ShapeDtypeStruct((n_nodes, 4, WQ), jnp.float32),
      mesh=_mesh,
      compiler_params=pltpu.CompilerParams(use_tc_tiling_on_sc=False),
      scratch_types=[
          pltpu.VMEM((U, CHUNK), jnp.int32),
          pltpu.VMEM((CHUNK, WQ), jnp.float32),
          pltpu.VMEM((ZR, WQ), jnp.float32),
          pltpu.VMEM_SHARED((acc_rows, WQ), jnp.float32),
      ],
  )
  def sc_count(dst_i, out, dstv, ones, zbuf, acc):
    c = lax.axis_index("c")
    s = lax.axis_index("s")

    _fill_rows(zbuf, ZR, 0.0)
    _fill_rows(ones, CHUNK, 1.0)
    for k in range(zpt):
      pltpu.sync_copy(zbuf, acc.at[pl.ds((s * zpt + k) * ZR, ZR)])
    plsc.subcore_barrier()

    cpt = nblk * U

    def body(b, _):
      row0 = s * cpt + b * U
      pltpu.sync_copy(dst_i.at[pl.ds(row0, U)], dstv)
      for j in range(U):
        pltpu.sync_copy(ones, acc.at[dstv.at[j]], add=True)
      return 0

    lax.fori_loop(0, nblk, body, 0)
    plsc.subcore_barrier()
    for p in range(2):
      pltpu.sync_copy(acc.at[pl.ds(s * fpt, fpt)],
                      out.at[pl.ds(s * fpt, fpt), 2 * c + p])

  return sc_count


_TCB = 4000
_PB = _TCB // 4


def _tc_proj0(xp, bdt, bdi):
  m = xp.shape[0]

  def body(x_ref, wt_ref, wi_ref, o_ref):
    xb = x_ref[...]
    o_ref[0] = jnp.dot(xb, wt_ref[...], preferred_element_type=jnp.float32)
    o_ref[1] = jnp.dot(xb, wi_ref[...], preferred_element_type=jnp.float32)

  return pl.pallas_call(
      body,
      grid=(m // _PB,),
      in_specs=[
          pl.BlockSpec((_PB, 512), lambda i: (i, 0)),
          pl.BlockSpec((512, 128), lambda i: (0, 0)),
          pl.BlockSpec((512, 128), lambda i: (0, 0)),
      ],
      out_specs=pl.BlockSpec((2, _PB, 128), lambda i: (0, i, 0)),
      out_shape=jax.ShapeDtypeStruct((2, m, 128), jnp.float32),
  )(xp, bdt, bdi)


def _tc_combine0(st, si, cntp, btp, bip):
  m = st.shape[0]

  def body(st_ref, si_ref, cnt_ref, bt_ref, bi_ref, o_ref):
    scale = 1.0 / jnp.maximum(cnt_ref[...], 1.0)
    o = st_ref[...] + si_ref[...] * scale + bt_ref[...] + bi_ref[...]
    o_ref[...] = jnp.maximum(o, 0.0)

  return pl.pallas_call(
      body,
      grid=(m // _PB,),
      in_specs=[
          pl.BlockSpec((_PB, 128), lambda i: (i, 0)),
          pl.BlockSpec((_PB, 128), lambda i: (i, 0)),
          pl.BlockSpec((_PB, 128), lambda i: (i, 0)),
          pl.BlockSpec((1, 128), lambda i: (0, 0)),
          pl.BlockSpec((1, 128), lambda i: (0, 0)),
      ],
      out_specs=pl.BlockSpec((_PB, 128), lambda i: (i, 0)),
      out_shape=jax.ShapeDtypeStruct((m, 128), jnp.float32),
  )(st, si, cntp, btp, bip)


def _tc_combine(st, si, cntp, bdt, bdi, btp, bip, hprev):
  m = st.shape[0]
  dcols = bdt.shape[1]
  residual = hprev is not None

  def body(st_ref, si_ref, cnt_ref, wt_ref, wi_ref, bt_ref, bi_ref, *rest):
    if residual:
      hp_ref, o_ref = rest
    else:
      (o_ref,) = rest
    scale = 1.0 / jnp.maximum(cnt_ref[...], 1.0)
    mi = si_ref[...] * scale
    o = (jnp.dot(st_ref[...], wt_ref[...], preferred_element_type=jnp.float32)
         + jnp.dot(mi, wi_ref[...], preferred_element_type=jnp.float32)
         + bt_ref[...] + bi_ref[...])
    if residual:
      o = o + hp_ref[...]
    o_ref[...] = jnp.maximum(o, 0.0)

  in_specs = [
      pl.BlockSpec((_PB, 128), lambda i: (i, 0)),
      pl.BlockSpec((_PB, 128), lambda i: (i, 0)),
      pl.BlockSpec((_PB, 128), lambda i: (i, 0)),
      pl.BlockSpec((128, dcols), lambda i: (0, 0)),
      pl.BlockSpec((128, dcols), lambda i: (0, 0)),
      pl.BlockSpec((1, dcols), lambda i: (0, 0)),
      pl.BlockSpec((1, dcols), lambda i: (0, 0)),
  ]
  args = [st, si, cntp, bdt, bdi, btp, bip]
  if residual:
    in_specs.append(pl.BlockSpec((_PB, 128), lambda i: (i, 0)))
    args.append(hprev)
  return pl.pallas_call(
      body,
      grid=(m // _PB,),
      in_specs=in_specs,
      out_specs=pl.BlockSpec((_PB, dcols), lambda i: (i, 0)),
      out_shape=jax.ShapeDtypeStruct((m, dcols), jnp.float32),
  )(*args)


def _prep_edges(src, dst, n_nodes, base):
  e = src.shape[0]
  blk = NS * U * CHUNK
  p = ((e + blk - 1) // blk) * blk
  pad = p - e
  fill = jnp.arange(pad, dtype=jnp.int32)
  src_p = jnp.concatenate([src, fill % n_nodes])
  dst_p = jnp.concatenate([dst, n_nodes + (fill % DUMP)])
  g = base + 4 * src_p
  gidx = jnp.stack([g, g + 1, g + 2, g + 3]).reshape(2, 2, p // CHUNK, CHUNK)
  return gidx, dst_p.reshape(p // CHUNK, CHUNK), p // blk


def _blockdiag4(w):
  din, dout = w.shape
  bd = jnp.zeros((4, din, 4, dout), jnp.float32)
  for k in range(4):
    bd = bd.at[k, :, k, :].set(w)
  return bd.reshape(4 * din, 4 * dout)


def kernel(x, edge_index_temp, edge_index_intersects, params):
  n = x.shape[0]
  src_t, dst_t = edge_index_temp[0], edge_index_temp[1]
  src_i, dst_i = edge_index_intersects[0], edge_index_intersects[1]

  gidx_t, dstp_t, nblk_t = _prep_edges(src_t, dst_t, n, 0)
  gidx_i, dstp_i, nblk_i = _prep_edges(src_i, dst_i, n, 0)
  gidx_i0 = gidx_i + 4 * n

  sc_count = _sc_count_fn(n, nblk_i)
  sc_layer0 = _sc_layer_fn(n, 8 * n, nblk_t, nblk_i)
  sc_layer = _sc_layer_fn(n, 4 * n, nblk_t, nblk_i)

  cntp = sc_count(dstp_i).reshape(n // 4, 128)

  p0 = params[0]
  xp = x.reshape(n // 4, 512)
  table0 = _tc_proj0(xp, _blockdiag4(p0["Wt"]),
                     _blockdiag4(p0["Wi"])).reshape(8 * n, WQ)
  st, si = sc_layer0(table0, gidx_t, dstp_t, gidx_i0, dstp_i)
  h = _tc_combine0(st.reshape(n // 4, 128), si.reshape(n // 4, 128), cntp,
                   jnp.tile(p0["bt"], 4).reshape(1, 128),
                   jnp.tile(p0["bi"], 4).reshape(1, 128))

  for p in params[1:]:
    st, si = sc_layer(h.reshape(4 * n, WQ), gidx_t, dstp_t, gidx_i, dstp_i)
    dout = p["Wt"].shape[1]
    hprev = h if dout == 32 else None
    h = _tc_combine(st.reshape(n // 4, 128), si.reshape(n // 4, 128), cntp,
                    _blockdiag4(p["Wt"]), _blockdiag4(p["Wi"]),
                    jnp.tile(p["bt"], 4).reshape(1, 4 * dout),
                    jnp.tile(p["bi"], 4).reshape(1, 4 * dout), hprev)
  return h.reshape(n, 64)


# speedup vs baseline: 12.3677x; 1.2005x over previous
import functools

import jax
import jax.numpy as jnp
from jax import lax
from jax.experimental import pallas as pl
from jax.experimental.pallas import tpu as pltpu
from jax.experimental.pallas import tpu_sc as plsc

NC = 2
NS = 16
WQ = 8
CHUNK = 128
U = 16
ZR = 1280
DUMP = 2400

_mesh = plsc.VectorSubcoreMesh(core_axis_name="c", subcore_axis_name="s",
                               num_cores=NC, num_subcores=NS)


def _fill_rows(ref, nrows, value):
  def body(k, _):
    ref[k] = jnp.full((WQ,), value, jnp.float32)
    return 0
  lax.fori_loop(0, nrows, body, 0, unroll=8)


def _sc_layer_fn(n_nodes, table_rows, nblk_t, nblk_i):
  acc_rows = n_nodes + DUMP
  assert acc_rows % (NS * ZR) == 0
  zpt = acc_rows // (NS * ZR)
  assert n_nodes % NS == 0
  fpt = n_nodes // NS

  @functools.partial(
      pl.kernel,
      out_type=[jax.ShapeDtypeStruct((n_nodes, 4, WQ), jnp.float32)] * 2,
      mesh=_mesh,
      compiler_params=pltpu.CompilerParams(use_tc_tiling_on_sc=False),
      scratch_types=[
          pltpu.VMEM((U, CHUNK), jnp.int32),
          pltpu.VMEM((U, CHUNK), jnp.int32),
          pltpu.VMEM((U, CHUNK, WQ), jnp.float32),
          pltpu.VMEM((ZR, WQ), jnp.float32),
          pltpu.VMEM_SHARED((acc_rows, WQ), jnp.float32),
          pltpu.SemaphoreType.DMA,
          pltpu.SemaphoreType.DMA,
      ],
  )
  def sc_layer(table, gidx_t, dst_t, gidx_i, dst_i, out_t, out_i,
               srcv, dstv, rows, zbuf, acc, sem, sem_sc):
    c = lax.axis_index("c")
    s = lax.axis_index("s")

    def zero_acc():
      for k in range(zpt):
        pltpu.sync_copy(zbuf, acc.at[pl.ds((s * zpt + k) * ZR, ZR)])

    def sweep(gidx, dst, nblk, p):
      cpt = nblk * U

      def drain_scatters():
        for j in range(U):
          pltpu.make_async_copy(rows.at[j], acc.at[dstv.at[j]],
                                sem_sc).wait()

      def body(b, _):
        row0 = s * cpt + b * U

        @pl.when(b > 0)
        def _():
          drain_scatters()

        pltpu.sync_copy(gidx.at[c, p, pl.ds(row0, U)], srcv)
        pltpu.sync_copy(dst.at[pl.ds(row0, U)], dstv)
        cps = [pltpu.async_copy(table.at[srcv.at[j]], rows.at[j], sem)
               for j in range(U)]
        for cp in cps:
          cp.wait()
        for j in range(U):
          pltpu.async_copy(rows.at[j], acc.at[dstv.at[j]], sem_sc, add=True)
        return 0

      lax.fori_loop(0, nblk, body, 0)
      drain_scatters()

    def quarter_pass(gidx, dst, nblk, p, out):
      zero_acc()
      plsc.subcore_barrier()
      sweep(gidx, dst, nblk, p)
      plsc.subcore_barrier()
      pltpu.sync_copy(acc.at[pl.ds(s * fpt, fpt)],
                      out.at[pl.ds(s * fpt, fpt), 2 * c + p])
      plsc.subcore_barrier()

    _fill_rows(zbuf, ZR, 0.0)
    for p in range(2):
      quarter_pass(gidx_t, dst_t, nblk_t, p, out_t)
    for p in range(2):
      quarter_pass(gidx_i, dst_i, nblk_i, p, out_i)

  return sc_layer


def _sc_count_fn(n_nodes, nblk):
  acc_rows = n_nodes + DUMP
  zpt = acc_rows // (NS * ZR)
  fpt = n_nodes // NS

  @functools.partial(
      pl.kernel,
      out_type=jax.ShapeDtypeStruct((n_nodes, 4, WQ), jnp.float32),
      mesh=_mesh,
      compiler_params=pltpu.CompilerParams(use_tc_tiling_on_sc=False),
      scratch_types=[
          pltpu.VMEM((U, CHUNK), jnp.int32),
          pltpu.VMEM((CHUNK, WQ), jnp.float32),
          pltpu.VMEM((ZR, WQ), jnp.float32),
          pltpu.VMEM_SHARED((acc_rows, WQ), jnp.float32),
      ],
  )
  def sc_count(dst_i, out, dstv, ones, zbuf, acc):
    c = lax.axis_index("c")
    s = lax.axis_index("s")

    _fill_rows(zbuf, ZR, 0.0)
    _fill_rows(ones, CHUNK, 1.0)
    for k in range(zpt):
      pltpu.sync_copy(zbuf, acc.at[pl.ds((s * zpt + k) * ZR, ZR)])
    plsc.subcore_barrier()

    cpt = nblk * U

    def body(b, _):
      row0 = s * cpt + b * U
      pltpu.sync_copy(dst_i.at[pl.ds(row0, U)], dstv)
      for j in range(U):
        pltpu.sync_copy(ones, acc.at[dstv.at[j]], add=True)
      return 0

    lax.fori_loop(0, nblk, body, 0)
    plsc.subcore_barrier()
    for p in range(2):
      pltpu.sync_copy(acc.at[pl.ds(s * fpt, fpt)],
                      out.at[pl.ds(s * fpt, fpt), 2 * c + p])

  return sc_count


_TCB = 4000
_PB = _TCB // 4


def _tc_proj0(xp, bdt, bdi):
  m = xp.shape[0]

  def body(x_ref, wt_ref, wi_ref, o_ref):
    xb = x_ref[...]
    o_ref[0] = jnp.dot(xb, wt_ref[...], preferred_element_type=jnp.float32)
    o_ref[1] = jnp.dot(xb, wi_ref[...], preferred_element_type=jnp.float32)

  return pl.pallas_call(
      body,
      grid=(m // _PB,),
      in_specs=[
          pl.BlockSpec((_PB, 512), lambda i: (i, 0)),
          pl.BlockSpec((512, 128), lambda i: (0, 0)),
          pl.BlockSpec((512, 128), lambda i: (0, 0)),
      ],
      out_specs=pl.BlockSpec((2, _PB, 128), lambda i: (0, i, 0)),
      out_shape=jax.ShapeDtypeStruct((2, m, 128), jnp.float32),
  )(xp, bdt, bdi)


def _tc_combine0(st, si, cntp, btp, bip):
  m = st.shape[0]

  def body(st_ref, si_ref, cnt_ref, bt_ref, bi_ref, o_ref):
    scale = 1.0 / jnp.maximum(cnt_ref[...], 1.0)
    o = st_ref[...] + si_ref[...] * scale + bt_ref[...] + bi_ref[...]
    o_ref[...] = jnp.maximum(o, 0.0)

  return pl.pallas_call(
      body,
      grid=(m // _PB,),
      in_specs=[
          pl.BlockSpec((_PB, 128), lambda i: (i, 0)),
          pl.BlockSpec((_PB, 128), lambda i: (i, 0)),
          pl.BlockSpec((_PB, 128), lambda i: (i, 0)),
          pl.BlockSpec((1, 128), lambda i: (0, 0)),
          pl.BlockSpec((1, 128), lambda i: (0, 0)),
      ],
      out_specs=pl.BlockSpec((_PB, 128), lambda i: (i, 0)),
      out_shape=jax.ShapeDtypeStruct((m, 128), jnp.float32),
  )(st, si, cntp, btp, bip)


def _tc_combine(st, si, cntp, bdt, bdi, btp, bip, hprev):
  m = st.shape[0]
  dcols = bdt.shape[1]
  residual = hprev is not None

  def body(st_ref, si_ref, cnt_ref, wt_ref, wi_ref, bt_ref, bi_ref, *rest):
    if residual:
      hp_ref, o_ref = rest
    else:
      (o_ref,) = rest
    scale = 1.0 / jnp.maximum(cnt_ref[...], 1.0)
    mi = si_ref[...] * scale
    o = (jnp.dot(st_ref[...], wt_ref[...], preferred_element_type=jnp.float32)
         + jnp.dot(mi, wi_ref[...], preferred_element_type=jnp.float32)
         + bt_ref[...] + bi_ref[...])
    if residual:
      o = o + hp_ref[...]
    o_ref[...] = jnp.maximum(o, 0.0)

  in_specs = [
      pl.BlockSpec((_PB, 128), lambda i: (i, 0)),
      pl.BlockSpec((_PB, 128), lambda i: (i, 0)),
      pl.BlockSpec((_PB, 128), lambda i: (i, 0)),
      pl.BlockSpec((128, dcols), lambda i: (0, 0)),
      pl.BlockSpec((128, dcols), lambda i: (0, 0)),
      pl.BlockSpec((1, dcols), lambda i: (0, 0)),
      pl.BlockSpec((1, dcols), lambda i: (0, 0)),
  ]
  args = [st, si, cntp, bdt, bdi, btp, bip]
  if residual:
    in_specs.append(pl.BlockSpec((_PB, 128), lambda i: (i, 0)))
    args.append(hprev)
  return pl.pallas_call(
      body,
      grid=(m // _PB,),
      in_specs=in_specs,
      out_specs=pl.BlockSpec((_PB, dcols), lambda i: (i, 0)),
      out_shape=jax.ShapeDtypeStruct((m, dcols), jnp.float32),
  )(*args)


def _prep_edges(src, dst, n_nodes, base):
  e = src.shape[0]
  blk = NS * U * CHUNK
  p = ((e + blk - 1) // blk) * blk
  pad = p - e
  fill = jnp.arange(pad, dtype=jnp.int32)
  src_p = jnp.concatenate([src, fill % n_nodes])
  dst_p = jnp.concatenate([dst, n_nodes + (fill % DUMP)])
  g = base + 4 * src_p
  gidx = jnp.stack([g, g + 1, g + 2, g + 3]).reshape(2, 2, p // CHUNK, CHUNK)
  return gidx, dst_p.reshape(p // CHUNK, CHUNK), p // blk


def _blockdiag4(w):
  din, dout = w.shape
  bd = jnp.zeros((4, din, 4, dout), jnp.float32)
  for k in range(4):
    bd = bd.at[k, :, k, :].set(w)
  return bd.reshape(4 * din, 4 * dout)


def kernel(x, edge_index_temp, edge_index_intersects, params):
  n = x.shape[0]
  src_t, dst_t = edge_index_temp[0], edge_index_temp[1]
  src_i, dst_i = edge_index_intersects[0], edge_index_intersects[1]

  gidx_t, dstp_t, nblk_t = _prep_edges(src_t, dst_t, n, 0)
  gidx_i, dstp_i, nblk_i = _prep_edges(src_i, dst_i, n, 0)
  gidx_i0 = gidx_i + 4 * n

  sc_count = _sc_count_fn(n, nblk_i)
  sc_layer0 = _sc_layer_fn(n, 8 * n, nblk_t, nblk_i)
  sc_layer = _sc_layer_fn(n, 4 * n, nblk_t, nblk_i)

  cntp = sc_count(dstp_i).reshape(n // 4, 128)

  p0 = params[0]
  xp = x.reshape(n // 4, 512)
  table0 = _tc_proj0(xp, _blockdiag4(p0["Wt"]),
                     _blockdiag4(p0["Wi"])).reshape(8 * n, WQ)
  st, si = sc_layer0(table0, gidx_t, dstp_t, gidx_i0, dstp_i)
  h = _tc_combine0(st.reshape(n // 4, 128), si.reshape(n // 4, 128), cntp,
                   jnp.tile(p0["bt"], 4).reshape(1, 128),
                   jnp.tile(p0["bi"], 4).reshape(1, 128))

  for p in params[1:]:
    st, si = sc_layer(h.reshape(4 * n, WQ), gidx_t, dstp_t, gidx_i, dstp_i)
    dout = p["Wt"].shape[1]
    hprev = h if dout == 32 else None
    h = _tc_combine(st.reshape(n // 4, 128), si.reshape(n // 4, 128), cntp,
                    _blockdiag4(p["Wt"]), _blockdiag4(p["Wi"]),
                    jnp.tile(p["bt"], 4).reshape(1, 4 * dout),
                    jnp.tile(p["bi"], 4).reshape(1, 4 * dout), hprev)
  return h.reshape(n, 64)


# speedup vs baseline: 13.8736x; 1.1218x over previous
import functools

import jax
import jax.numpy as jnp
from jax import lax
from jax.experimental import pallas as pl
from jax.experimental.pallas import tpu as pltpu
from jax.experimental.pallas import tpu_sc as plsc

NC = 2
NS = 16
WQ = 8
CHUNK = 128
U = 32
ZR = 1280
DUMP = 2400

_mesh = plsc.VectorSubcoreMesh(core_axis_name="c", subcore_axis_name="s",
                               num_cores=NC, num_subcores=NS)


def _fill_rows(ref, nrows, value):
  def body(k, _):
    ref[k] = jnp.full((WQ,), value, jnp.float32)
    return 0
  lax.fori_loop(0, nrows, body, 0, unroll=8)


def _sc_layer_fn(n_nodes, table_rows, nblk_t, nblk_i):
  acc_rows = n_nodes + DUMP
  assert acc_rows % (NS * ZR) == 0
  zpt = acc_rows // (NS * ZR)
  assert n_nodes % NS == 0
  fpt = n_nodes // NS

  @functools.partial(
      pl.kernel,
      out_type=[jax.ShapeDtypeStruct((n_nodes, 4, WQ), jnp.float32)] * 2,
      mesh=_mesh,
      compiler_params=pltpu.CompilerParams(use_tc_tiling_on_sc=False),
      scratch_types=[
          pltpu.VMEM((U, CHUNK), jnp.int32),
          pltpu.VMEM((U, CHUNK), jnp.int32),
          pltpu.VMEM((U, CHUNK, WQ), jnp.float32),
          pltpu.VMEM((ZR, WQ), jnp.float32),
          pltpu.VMEM_SHARED((acc_rows, WQ), jnp.float32),
          pltpu.SemaphoreType.DMA,
          pltpu.SemaphoreType.DMA,
      ],
  )
  def sc_layer(table, gidx_t, dst_t, gidx_i, dst_i, out_t, out_i,
               srcv, dstv, rows, zbuf, acc, sem, sem_sc):
    c = lax.axis_index("c")
    s = lax.axis_index("s")

    def zero_acc():
      for k in range(zpt):
        pltpu.sync_copy(zbuf, acc.at[pl.ds((s * zpt + k) * ZR, ZR)])

    def sweep(gidx, dst, nblk, p):
      cpt = nblk * U

      def drain_scatters():
        for j in range(U):
          pltpu.make_async_copy(rows.at[j], acc.at[dstv.at[j]],
                                sem_sc).wait()

      def body(b, _):
        row0 = s * cpt + b * U

        @pl.when(b > 0)
        def _():
          drain_scatters()

        pltpu.sync_copy(gidx.at[c, p, pl.ds(row0, U)], srcv)
        pltpu.sync_copy(dst.at[pl.ds(row0, U)], dstv)
        cps = [pltpu.async_copy(table.at[srcv.at[j]], rows.at[j], sem)
               for j in range(U)]
        for cp in cps:
          cp.wait()
        for j in range(U):
          pltpu.async_copy(rows.at[j], acc.at[dstv.at[j]], sem_sc, add=True)
        return 0

      lax.fori_loop(0, nblk, body, 0)
      drain_scatters()

    def quarter_pass(gidx, dst, nblk, p, out):
      zero_acc()
      plsc.subcore_barrier()
      sweep(gidx, dst, nblk, p)
      plsc.subcore_barrier()
      pltpu.sync_copy(acc.at[pl.ds(s * fpt, fpt)],
                      out.at[pl.ds(s * fpt, fpt), 2 * c + p])
      plsc.subcore_barrier()

    _fill_rows(zbuf, ZR, 0.0)
    for p in range(2):
      quarter_pass(gidx_t, dst_t, nblk_t, p, out_t)
    for p in range(2):
      quarter_pass(gidx_i, dst_i, nblk_i, p, out_i)

  return sc_layer


def _sc_count_fn(n_nodes, nblk):
  acc_rows = n_nodes + DUMP
  zpt = acc_rows // (NS * ZR)
  fpt = n_nodes // NS

  @functools.partial(
      pl.kernel,
      out_type=jax.ShapeDtypeStruct((n_nodes, 4, WQ), jnp.float32),
      mesh=_mesh,
      compiler_params=pltpu.CompilerParams(use_tc_tiling_on_sc=False),
      scratch_types=[
          pltpu.VMEM((U, CHUNK), jnp.int32),
          pltpu.VMEM((CHUNK, WQ), jnp.float32),
          pltpu.VMEM((ZR, WQ), jnp.float32),
          pltpu.VMEM_SHARED((acc_rows, WQ), jnp.float32),
      ],
  )
  def sc_count(dst_i, out, dstv, ones, zbuf, acc):
    c = lax.axis_index("c")
    s = lax.axis_index("s")

    _fill_rows(zbuf, ZR, 0.0)
    _fill_rows(ones, CHUNK, 1.0)
    for k in range(zpt):
      pltpu.sync_copy(zbuf, acc.at[pl.ds((s * zpt + k) * ZR, ZR)])
    plsc.subcore_barrier()

    cpt = nblk * U

    def body(b, _):
      row0 = s * cpt + b * U
      pltpu.sync_copy(dst_i.at[pl.ds(row0, U)], dstv)
      for j in range(U):
        pltpu.sync_copy(ones, acc.at[dstv.at[j]], add=True)
      return 0

    lax.fori_loop(0, nblk, body, 0)
    plsc.subcore_barrier()
    for p in range(2):
      pltpu.sync_copy(acc.at[pl.ds(s * fpt, fpt)],
                      out.at[pl.ds(s * fpt, fpt), 2 * c + p])

  return sc_count


_TCB = 4000
_PB = _TCB // 4


def _tc_proj0(xp, bdt, bdi):
  m = xp.shape[0]

  def body(x_ref, wt_ref, wi_ref, o_ref):
    xb = x_ref[...]
    o_ref[0] = jnp.dot(xb, wt_ref[...], preferred_element_type=jnp.float32)
    o_ref[1] = jnp.dot(xb, wi_ref[...], preferred_element_type=jnp.float32)

  return pl.pallas_call(
      body,
      grid=(m // _PB,),
      in_specs=[
          pl.BlockSpec((_PB, 512), lambda i: (i, 0)),
          pl.BlockSpec((512, 128), lambda i: (0, 0)),
          pl.BlockSpec((512, 128), lambda i: (0, 0)),
      ],
      out_specs=pl.BlockSpec((2, _PB, 128), lambda i: (0, i, 0)),
      out_shape=jax.ShapeDtypeStruct((2, m, 128), jnp.float32),
  )(xp, bdt, bdi)


def _tc_combine0(st, si, cntp, btp, bip):
  m = st.shape[0]

  def body(st_ref, si_ref, cnt_ref, bt_ref, bi_ref, o_ref):
    scale = 1.0 / jnp.maximum(cnt_ref[...], 1.0)
    o = st_ref[...] + si_ref[...] * scale + bt_ref[...] + bi_ref[...]
    o_ref[...] = jnp.maximum(o, 0.0)

  return pl.pallas_call(
      body,
      grid=(m // _PB,),
      in_specs=[
          pl.BlockSpec((_PB, 128), lambda i: (i, 0)),
          pl.BlockSpec((_PB, 128), lambda i: (i, 0)),
          pl.BlockSpec((_PB, 128), lambda i: (i, 0)),
          pl.BlockSpec((1, 128), lambda i: (0, 0)),
          pl.BlockSpec((1, 128), lambda i: (0, 0)),
      ],
      out_specs=pl.BlockSpec((_PB, 128), lambda i: (i, 0)),
      out_shape=jax.ShapeDtypeStruct((m, 128), jnp.float32),
  )(st, si, cntp, btp, bip)


def _tc_combine(st, si, cntp, bdt, bdi, btp, bip, hprev):
  m = st.shape[0]
  dcols = bdt.shape[1]
  residual = hprev is not None

  def body(st_ref, si_ref, cnt_ref, wt_ref, wi_ref, bt_ref, bi_ref, *rest):
    if residual:
      hp_ref, o_ref = rest
    else:
      (o_ref,) = rest
    scale = 1.0 / jnp.maximum(cnt_ref[...], 1.0)
    mi = si_ref[...] * scale
    o = (jnp.dot(st_ref[...], wt_ref[...], preferred_element_type=jnp.float32)
         + jnp.dot(mi, wi_ref[...], preferred_element_type=jnp.float32)
         + bt_ref[...] + bi_ref[...])
    if residual:
      o = o + hp_ref[...]
    o_ref[...] = jnp.maximum(o, 0.0)

  in_specs = [
      pl.BlockSpec((_PB, 128), lambda i: (i, 0)),
      pl.BlockSpec((_PB, 128), lambda i: (i, 0)),
      pl.BlockSpec((_PB, 128), lambda i: (i, 0)),
      pl.BlockSpec((128, dcols), lambda i: (0, 0)),
      pl.BlockSpec((128, dcols), lambda i: (0, 0)),
      pl.BlockSpec((1, dcols), lambda i: (0, 0)),
      pl.BlockSpec((1, dcols), lambda i: (0, 0)),
  ]
  args = [st, si, cntp, bdt, bdi, btp, bip]
  if residual:
    in_specs.append(pl.BlockSpec((_PB, 128), lambda i: (i, 0)))
    args.append(hprev)
  return pl.pallas_call(
      body,
      grid=(m // _PB,),
      in_specs=in_specs,
      out_specs=pl.BlockSpec((_PB, dcols), lambda i: (i, 0)),
      out_shape=jax.ShapeDtypeStruct((m, dcols), jnp.float32),
  )(*args)


def _prep_edges(src, dst, n_nodes, base):
  e = src.shape[0]
  blk = NS * U * CHUNK
  p = ((e + blk - 1) // blk) * blk
  pad = p - e
  fill = jnp.arange(pad, dtype=jnp.int32)
  src_p = jnp.concatenate([src, fill % n_nodes])
  dst_p = jnp.concatenate([dst, n_nodes + (fill % DUMP)])
  g = base + 4 * src_p
  gidx = jnp.stack([g, g + 1, g + 2, g + 3]).reshape(2, 2, p // CHUNK, CHUNK)
  return gidx, dst_p.reshape(p // CHUNK, CHUNK), p // blk


def _blockdiag4(w):
  din, dout = w.shape
  bd = jnp.zeros((4, din, 4, dout), jnp.float32)
  for k in range(4):
    bd = bd.at[k, :, k, :].set(w)
  return bd.reshape(4 * din, 4 * dout)


def kernel(x, edge_index_temp, edge_index_intersects, params):
  n = x.shape[0]
  src_t, dst_t = edge_index_temp[0], edge_index_temp[1]
  src_i, dst_i = edge_index_intersects[0], edge_index_intersects[1]

  gidx_t, dstp_t, nblk_t = _prep_edges(src_t, dst_t, n, 0)
  gidx_i, dstp_i, nblk_i = _prep_edges(src_i, dst_i, n, 0)
  gidx_i0 = gidx_i + 4 * n

  sc_count = _sc_count_fn(n, nblk_i)
  sc_layer0 = _sc_layer_fn(n, 8 * n, nblk_t, nblk_i)
  sc_layer = _sc_layer_fn(n, 4 * n, nblk_t, nblk_i)

  cntp = sc_count(dstp_i).reshape(n // 4, 128)

  p0 = params[0]
  xp = x.reshape(n // 4, 512)
  table0 = _tc_proj0(xp, _blockdiag4(p0["Wt"]),
                     _blockdiag4(p0["Wi"])).reshape(8 * n, WQ)
  st, si = sc_layer0(table0, gidx_t, dstp_t, gidx_i0, dstp_i)
  h = _tc_combine0(st.reshape(n // 4, 128), si.reshape(n // 4, 128), cntp,
                   jnp.tile(p0["bt"], 4).reshape(1, 128),
                   jnp.tile(p0["bi"], 4).reshape(1, 128))

  for p in params[1:]:
    st, si = sc_layer(h.reshape(4 * n, WQ), gidx_t, dstp_t, gidx_i, dstp_i)
    dout = p["Wt"].shape[1]
    hprev = h if dout == 32 else None
    h = _tc_combine(st.reshape(n // 4, 128), si.reshape(n // 4, 128), cntp,
                    _blockdiag4(p["Wt"]), _blockdiag4(p["Wi"]),
                    jnp.tile(p["bt"], 4).reshape(1, 4 * dout),
                    jnp.tile(p["bi"], 4).reshape(1, 4 * dout), hprev)
  return h.reshape(n, 64)


# speedup vs baseline: 17.5689x; 1.2664x over previous
import functools

import jax
import jax.numpy as jnp
from jax import lax
from jax.experimental import pallas as pl
from jax.experimental.pallas import tpu as pltpu
from jax.experimental.pallas import tpu_sc as plsc

NC = 2
NS = 16
WH = 16
CHUNK = 128
U = 8
ZR = 256
DUMP = 2400

_mesh = plsc.VectorSubcoreMesh(core_axis_name="c", subcore_axis_name="s",
                               num_cores=NC, num_subcores=NS)


def _fill_rows(ref, nrows, value):
  def body(k, _):
    ref[k] = jnp.full((WH,), value, jnp.float32)
    return 0
  lax.fori_loop(0, nrows, body, 0, unroll=8)


def _sc_layer_fn(n_nodes, table_rows, nblk_t, nblk_i):
  acc_rows = n_nodes + DUMP
  assert acc_rows % (NS * ZR) == 0
  zpt = acc_rows // (NS * ZR)
  assert n_nodes % NS == 0
  fpt = n_nodes // NS

  @functools.partial(
      pl.kernel,
      out_type=[jax.ShapeDtypeStruct((n_nodes, NC, WH), jnp.float32)] * 2,
      mesh=_mesh,
      compiler_params=pltpu.CompilerParams(use_tc_tiling_on_sc=False),
      scratch_types=[
          pltpu.VMEM((U, CHUNK), jnp.int32),
          pltpu.VMEM((U, CHUNK), jnp.int32),
          pltpu.VMEM((U, CHUNK, WH), jnp.float32),
          pltpu.VMEM((ZR, WH), jnp.float32),
          pltpu.VMEM_SHARED((acc_rows, WH), jnp.float32),
          pltpu.SemaphoreType.DMA,
          pltpu.SemaphoreType.DMA,
      ],
  )
  def sc_layer(table, gidx_t, dst_t, gidx_i, dst_i, out_t, out_i,
               srcv, dstv, rows, zbuf, acc, sem, sem_sc):
    c = lax.axis_index("c")
    s = lax.axis_index("s")

    def zero_acc():
      for k in range(zpt):
        pltpu.sync_copy(zbuf, acc.at[pl.ds((s * zpt + k) * ZR, ZR)])

    def sweep(gidx, dst, nblk):
      cpt = nblk * U

      def drain_scatters():
        for j in range(U):
          pltpu.make_async_copy(rows.at[j], acc.at[dstv.at[j]],
                                sem_sc).wait()

      def body(b, _):
        row0 = s * cpt + b * U

        @pl.when(b > 0)
        def _():
          drain_scatters()

        pltpu.sync_copy(gidx.at[c, pl.ds(row0, U)], srcv)
        pltpu.sync_copy(dst.at[pl.ds(row0, U)], dstv)
        cps = [pltpu.async_copy(table.at[srcv.at[j]], rows.at[j], sem)
               for j in range(U)]
        for cp in cps:
          cp.wait()
        for j in range(U):
          pltpu.async_copy(rows.at[j], acc.at[dstv.at[j]], sem_sc, add=True)
        return 0

      lax.fori_loop(0, nblk, body, 0)
      drain_scatters()

    def half_pass(gidx, dst, nblk, out):
      zero_acc()
      plsc.subcore_barrier()
      sweep(gidx, dst, nblk)
      plsc.subcore_barrier()
      pltpu.sync_copy(acc.at[pl.ds(s * fpt, fpt)],
                      out.at[pl.ds(s * fpt, fpt), c])
      plsc.subcore_barrier()

    _fill_rows(zbuf, ZR, 0.0)
    half_pass(gidx_t, dst_t, nblk_t, out_t)
    half_pass(gidx_i, dst_i, nblk_i, out_i)

  return sc_layer


def _sc_count_fn(n_nodes, nblk):
  acc_rows = n_nodes + DUMP
  zpt = acc_rows // (NS * ZR)
  fpt = n_nodes // NS

  @functools.partial(
      pl.kernel,
      out_type=jax.ShapeDtypeStruct((n_nodes, NC, WH), jnp.float32),
      mesh=_mesh,
      compiler_params=pltpu.CompilerParams(use_tc_tiling_on_sc=False),
      scratch_types=[
          pltpu.VMEM((U, CHUNK), jnp.int32),
          pltpu.VMEM((CHUNK, WH), jnp.float32),
          pltpu.VMEM((ZR, WH), jnp.float32),
          pltpu.VMEM_SHARED((acc_rows, WH), jnp.float32),
      ],
  )
  def sc_count(dst_i, out, dstv, ones, zbuf, acc):
    c = lax.axis_index("c")
    s = lax.axis_index("s")

    _fill_rows(zbuf, ZR, 0.0)
    _fill_rows(ones, CHUNK, 1.0)
    for k in range(zpt):
      pltpu.sync_copy(zbuf, acc.at[pl.ds((s * zpt + k) * ZR, ZR)])
    plsc.subcore_barrier()

    cpt = nblk * U

    def body(b, _):
      row0 = s * cpt + b * U
      pltpu.sync_copy(dst_i.at[pl.ds(row0, U)], dstv)
      for j in range(U):
        pltpu.sync_copy(ones, acc.at[dstv.at[j]], add=True)
      return 0

    lax.fori_loop(0, nblk, body, 0)
    plsc.subcore_barrier()
    pltpu.sync_copy(acc.at[pl.ds(s * fpt, fpt)],
                    out.at[pl.ds(s * fpt, fpt), c])

  return sc_count


_TCB = 4000
_PB = _TCB // 4


def _tc_proj0(xp, bdt, bdi):
  m = xp.shape[0]

  def body(x_ref, wt_ref, wi_ref, o_ref):
    xb = x_ref[...]
    o_ref[0] = jnp.dot(xb, wt_ref[...], preferred_element_type=jnp.float32)
    o_ref[1] = jnp.dot(xb, wi_ref[...], preferred_element_type=jnp.float32)

  return pl.pallas_call(
      body,
      grid=(m // _PB,),
      in_specs=[
          pl.BlockSpec((_PB, 512), lambda i: (i, 0)),
          pl.BlockSpec((512, 128), lambda i: (0, 0)),
          pl.BlockSpec((512, 128), lambda i: (0, 0)),
      ],
      out_specs=pl.BlockSpec((2, _PB, 128), lambda i: (0, i, 0)),
      out_shape=jax.ShapeDtypeStruct((2, m, 128), jnp.float32),
  )(xp, bdt, bdi)


def _tc_combine0(st, si, cntp, btp, bip):
  m = st.shape[0]

  def body(st_ref, si_ref, cnt_ref, bt_ref, bi_ref, o_ref):
    scale = 1.0 / jnp.maximum(cnt_ref[...], 1.0)
    o = st_ref[...] + si_ref[...] * scale + bt_ref[...] + bi_ref[...]
    o_ref[...] = jnp.maximum(o, 0.0)

  return pl.pallas_call(
      body,
      grid=(m // _PB,),
      in_specs=[
          pl.BlockSpec((_PB, 128), lambda i: (i, 0)),
          pl.BlockSpec((_PB, 128), lambda i: (i, 0)),
          pl.BlockSpec((_PB, 128), lambda i: (i, 0)),
          pl.BlockSpec((1, 128), lambda i: (0, 0)),
          pl.BlockSpec((1, 128), lambda i: (0, 0)),
      ],
      out_specs=pl.BlockSpec((_PB, 128), lambda i: (i, 0)),
      out_shape=jax.ShapeDtypeStruct((m, 128), jnp.float32),
  )(st, si, cntp, btp, bip)


def _tc_combine(st, si, cntp, bdt, bdi, btp, bip, hprev):
  m = st.shape[0]
  dcols = bdt.shape[1]
  residual = hprev is not None

  def body(st_ref, si_ref, cnt_ref, wt_ref, wi_ref, bt_ref, bi_ref, *rest):
    if residual:
      hp_ref, o_ref = rest
    else:
      (o_ref,) = rest
    scale = 1.0 / jnp.maximum(cnt_ref[...], 1.0)
    mi = si_ref[...] * scale
    o = (jnp.dot(st_ref[...], wt_ref[...], preferred_element_type=jnp.float32)
         + jnp.dot(mi, wi_ref[...], preferred_element_type=jnp.float32)
         + bt_ref[...] + bi_ref[...])
    if residual:
      o = o + hp_ref[...]
    o_ref[...] = jnp.maximum(o, 0.0)

  in_specs = [
      pl.BlockSpec((_PB, 128), lambda i: (i, 0)),
      pl.BlockSpec((_PB, 128), lambda i: (i, 0)),
      pl.BlockSpec((_PB, 128), lambda i: (i, 0)),
      pl.BlockSpec((128, dcols), lambda i: (0, 0)),
      pl.BlockSpec((128, dcols), lambda i: (0, 0)),
      pl.BlockSpec((1, dcols), lambda i: (0, 0)),
      pl.BlockSpec((1, dcols), lambda i: (0, 0)),
  ]
  args = [st, si, cntp, bdt, bdi, btp, bip]
  if residual:
    in_specs.append(pl.BlockSpec((_PB, 128), lambda i: (i, 0)))
    args.append(hprev)
  return pl.pallas_call(
      body,
      grid=(m // _PB,),
      in_specs=in_specs,
      out_specs=pl.BlockSpec((_PB, dcols), lambda i: (i, 0)),
      out_shape=jax.ShapeDtypeStruct((m, dcols), jnp.float32),
  )(*args)


def _prep_edges(src, dst, n_nodes, base):
  e = src.shape[0]
  blk = NS * U * CHUNK
  p = ((e + blk - 1) // blk) * blk
  pad = p - e
  fill = jnp.arange(pad, dtype=jnp.int32)
  src_p = jnp.concatenate([src, fill % n_nodes])
  dst_p = jnp.concatenate([dst, n_nodes + (fill % DUMP)])
  g = base + 2 * src_p
  gidx = jnp.stack([g, g + 1]).reshape(2, p // CHUNK, CHUNK)
  return gidx, dst_p.reshape(p // CHUNK, CHUNK), p // blk


def _blockdiag4(w):
  din, dout = w.shape
  bd = jnp.zeros((4, din, 4, dout), jnp.float32)
  for k in range(4):
    bd = bd.at[k, :, k, :].set(w)
  return bd.reshape(4 * din, 4 * dout)


def kernel(x, edge_index_temp, edge_index_intersects, params):
  n = x.shape[0]
  src_t, dst_t = edge_index_temp[0], edge_index_temp[1]
  src_i, dst_i = edge_index_intersects[0], edge_index_intersects[1]

  gidx_t, dstp_t, nblk_t = _prep_edges(src_t, dst_t, n, 0)
  gidx_i, dstp_i, nblk_i = _prep_edges(src_i, dst_i, n, 0)
  gidx_i0 = gidx_i + 2 * n

  sc_count = _sc_count_fn(n, nblk_i)
  sc_layer0 = _sc_layer_fn(n, 4 * n, nblk_t, nblk_i)
  sc_layer = _sc_layer_fn(n, 2 * n, nblk_t, nblk_i)

  cntp = sc_count(dstp_i).reshape(n // 4, 128)

  p0 = params[0]
  xp = x.reshape(n // 4, 512)
  table0 = _tc_proj0(xp, _blockdiag4(p0["Wt"]),
                     _blockdiag4(p0["Wi"])).reshape(4 * n, WH)
  st, si = sc_layer0(table0, gidx_t, dstp_t, gidx_i0, dstp_i)
  h = _tc_combine0(st.reshape(n // 4, 128), si.reshape(n // 4, 128), cntp,
                   jnp.tile(p0["bt"], 4).reshape(1, 128),
                   jnp.tile(p0["bi"], 4).reshape(1, 128))

  for p in params[1:]:
    st, si = sc_layer(h.reshape(2 * n, WH), gidx_t, dstp_t, gidx_i, dstp_i)
    dout = p["Wt"].shape[1]
    hprev = h if dout == 32 else None
    h = _tc_combine(st.reshape(n // 4, 128), si.reshape(n // 4, 128), cntp,
                    _blockdiag4(p["Wt"]), _blockdiag4(p["Wi"]),
                    jnp.tile(p["bt"], 4).reshape(1, 4 * dout),
                    jnp.tile(p["bi"], 4).reshape(1, 4 * dout), hprev)
  return h.reshape(n, 64)
